# Initial kernel scaffold; baseline (speedup 1.0000x reference)
import jax
import jax.numpy as jnp
from jax.experimental import pallas as pl


def kernel(inputs, w_C, w_B_C, w_A_B):
    raise NotImplementedError("write your pallas kernel here")


# speedup vs baseline: 2.5500x; 2.5500x over previous
import functools

import jax
import jax.numpy as jnp
from jax import lax
from jax.experimental import pallas as pl
from jax.experimental.pallas import tpu as pltpu
from jax.experimental.pallas import tpu_sc as plsc

N = 1000
BQ = 16384
NC, NS, L = 2, 16, 16
NW = NC * NS
BPW = BQ // NW
CHUNK = 128
NCHUNK = BPW // CHUNK


def _lse_body(wc_ref, wbc_ref, wab_ref, tc_ref, tb_ref):
    wbc = wbc_ref[...]
    m1 = jnp.max(wbc, axis=1, keepdims=True)
    lse_bc = m1 + jnp.log(jnp.sum(jnp.exp(wbc - m1), axis=1, keepdims=True))
    wab = wab_ref[...]
    m2 = jnp.max(wab, axis=1, keepdims=True)
    lse_ab = m2 + jnp.log(jnp.sum(jnp.exp(wab - m2), axis=1, keepdims=True))
    wc = wc_ref[...]
    mc = jnp.max(wc)
    lse_c = mc + jnp.log(jnp.sum(jnp.exp(wc - mc)))
    tc_ref[...] = wc - lse_c - lse_bc
    tb_ref[...] = -lse_ab


_lse_call = pl.pallas_call(
    _lse_body,
    out_shape=(
        jax.ShapeDtypeStruct((N, 1), jnp.float32),
        jax.ShapeDtypeStruct((N, 1), jnp.float32),
    ),
)


@functools.cache
def _build_gather_combine():
  mesh = plsc.VectorSubcoreMesh(core_axis_name="c", subcore_axis_name="s")

  @functools.partial(
      pl.kernel,
      out_type=jax.ShapeDtypeStruct((BQ,), jnp.float32),
      mesh=mesh,
      scratch_types=[
          pltpu.VMEM((BPW,), jnp.int32),
          pltpu.VMEM((BPW,), jnp.int32),
          pltpu.VMEM((BPW,), jnp.int32),
          pltpu.VMEM((NCHUNK, CHUNK), jnp.int32),
          pltpu.VMEM((NCHUNK, CHUNK), jnp.int32),
          pltpu.VMEM((NCHUNK, CHUNK), jnp.int32),
          pltpu.VMEM((NCHUNK, CHUNK), jnp.int32),
          pltpu.VMEM((BPW,), jnp.float32),
          pltpu.VMEM((BPW,), jnp.float32),
          pltpu.VMEM((BPW,), jnp.float32),
          pltpu.VMEM((BPW,), jnp.float32),
          pltpu.VMEM((BPW,), jnp.float32),
          pltpu.SemaphoreType.DMA,
      ],
  )
  def _gather_combine(ia_hbm, ib_hbm, ic_hbm, fbc_hbm, fab_hbm, tc_hbm, tb_hbm,
                      out_hbm, ia_v, ib_v, ic_v, xbc_v, xab_v, xc_v, xb_v,
                      gbc_v, gab_v, gtc_v, gtb_v, out_v, sem):
    wid = lax.axis_index("s") * NC + lax.axis_index("c")
    base = wid * BPW
    pltpu.sync_copy(ia_hbm.at[pl.ds(base, BPW)], ia_v)
    pltpu.sync_copy(ib_hbm.at[pl.ds(base, BPW)], ib_v)
    pltpu.sync_copy(ic_hbm.at[pl.ds(base, BPW)], ic_v)
    for j in range(BPW // L):
        r, o = j // (CHUNK // L), (j % (CHUNK // L)) * L
        sl = pl.ds(j * L, L)
        a16 = ia_v[sl]
        b16 = ib_v[sl]
        c16 = ic_v[sl]
        osl = pl.ds(o, L)
        xbc_v[r, osl] = c16 * N + b16
        xab_v[r, osl] = b16 * N + a16
        xc_v[r, osl] = c16
        xb_v[r, osl] = b16
    copies = []
    for r in range(NCHUNK):
        dsl = pl.ds(r * CHUNK, CHUNK)
        copies.append(pltpu.async_copy(
            fbc_hbm.at[xbc_v.at[r]], gbc_v.at[dsl], sem))
        copies.append(pltpu.async_copy(
            fab_hbm.at[xab_v.at[r]], gab_v.at[dsl], sem))
        copies.append(pltpu.async_copy(
            tc_hbm.at[xc_v.at[r]], gtc_v.at[dsl], sem))
        copies.append(pltpu.async_copy(
            tb_hbm.at[xb_v.at[r]], gtb_v.at[dsl], sem))
    for cp in copies:
        cp.wait()
    for j in range(BPW // L):
        sl = pl.ds(j * L, L)
        out_v[sl] = (gbc_v[sl] + gab_v[sl]) + (gtc_v[sl] + gtb_v[sl])
    pltpu.sync_copy(out_v, out_hbm.at[pl.ds(base, BPW)])

  return _gather_combine


def kernel(inputs, w_C, w_B_C, w_A_B):
    idx = inputs.astype(jnp.int32)
    ia, ib, ic = idx[:, 0], idx[:, 1], idx[:, 2]
    tc2d, tb2d = _lse_call(w_C.reshape(N, 1), w_B_C, w_A_B)
    fbc = w_B_C.reshape(-1)
    fab = w_A_B.reshape(-1)
    return _build_gather_combine()(ia, ib, ic, fbc, fab,
                                   tc2d.reshape(N), tb2d.reshape(N))


# speedup vs baseline: 3.5189x; 1.3799x over previous
import functools

import jax
import jax.numpy as jnp
from jax import lax
from jax.experimental import pallas as pl
from jax.experimental.pallas import tpu as pltpu
from jax.experimental.pallas import tpu_sc as plsc

N = 1000
NP = 1024
BQ = 16384
NC, NS, L = 2, 16, 16
NW = NC * NS
BPW = BQ // NW
CHUNK = 128
NCHUNK = BPW // CHUNK


def _fold_body(wc_ref, wbc_ref, wab_ref, fbc_ref, fab_ref):
    wbc = wbc_ref[...]
    m1 = jnp.max(wbc, axis=1, keepdims=True)
    lse_bc = m1 + jnp.log(jnp.sum(jnp.exp(wbc - m1), axis=1, keepdims=True))
    wab = wab_ref[...]
    m2 = jnp.max(wab, axis=1, keepdims=True)
    lse_ab = m2 + jnp.log(jnp.sum(jnp.exp(wab - m2), axis=1, keepdims=True))
    wc = wc_ref[...]
    mc = jnp.max(wc)
    lse_c = mc + jnp.log(jnp.sum(jnp.exp(wc - mc)))
    fbc_ref[:, :N] = wbc + (wc - lse_c - lse_bc)
    fab_ref[:, :N] = wab - lse_ab


_fold_call = pl.pallas_call(
    _fold_body,
    out_shape=(
        jax.ShapeDtypeStruct((N, NP), jnp.float32),
        jax.ShapeDtypeStruct((N, NP), jnp.float32),
    ),
)


@functools.cache
def _build_gather_combine():
  mesh = plsc.VectorSubcoreMesh(core_axis_name="c", subcore_axis_name="s")

  @functools.partial(
      pl.kernel,
      out_type=jax.ShapeDtypeStruct((BQ,), jnp.float32),
      mesh=mesh,
      scratch_types=[
          pltpu.VMEM((BPW,), jnp.int32),
          pltpu.VMEM((BPW,), jnp.int32),
          pltpu.VMEM((BPW,), jnp.int32),
          pltpu.VMEM((NCHUNK, CHUNK), jnp.int32),
          pltpu.VMEM((NCHUNK, CHUNK), jnp.int32),
          pltpu.VMEM((BPW,), jnp.float32),
          pltpu.VMEM((BPW,), jnp.float32),
          pltpu.VMEM((BPW,), jnp.float32),
          pltpu.SemaphoreType.DMA,
      ],
  )
  def _gather_combine(ia_hbm, ib_hbm, ic_hbm, fbc_hbm, fab_hbm,
                      out_hbm, ia_v, ib_v, ic_v, xbc_v, xab_v,
                      gbc_v, gab_v, out_v, sem):
    wid = lax.axis_index("s") * NC + lax.axis_index("c")
    base = wid * BPW
    pltpu.sync_copy(ia_hbm.at[pl.ds(base, BPW)], ia_v)
    pltpu.sync_copy(ib_hbm.at[pl.ds(base, BPW)], ib_v)
    pltpu.sync_copy(ic_hbm.at[pl.ds(base, BPW)], ic_v)
    for j in range(BPW // L):
        r, o = j // (CHUNK // L), (j % (CHUNK // L)) * L
        sl = pl.ds(j * L, L)
        a16 = ia_v[sl]
        b16 = ib_v[sl]
        c16 = ic_v[sl]
        osl = pl.ds(o, L)
        xbc_v[r, osl] = c16 * NP + b16
        xab_v[r, osl] = b16 * NP + a16
    copies = []
    for r in range(NCHUNK):
        dsl = pl.ds(r * CHUNK, CHUNK)
        copies.append(pltpu.async_copy(
            fbc_hbm.at[xbc_v.at[r]], gbc_v.at[dsl], sem))
        copies.append(pltpu.async_copy(
            fab_hbm.at[xab_v.at[r]], gab_v.at[dsl], sem))
    for cp in copies:
        cp.wait()
    for j in range(BPW // L):
        sl = pl.ds(j * L, L)
        out_v[sl] = gbc_v[sl] + gab_v[sl]
    pltpu.sync_copy(out_v, out_hbm.at[pl.ds(base, BPW)])

  return _gather_combine


def kernel(inputs, w_C, w_B_C, w_A_B):
    idx = inputs.astype(jnp.int32)
    ia, ib, ic = idx[:, 0], idx[:, 1], idx[:, 2]
    fbc2, fab2 = _fold_call(w_C.reshape(N, 1), w_B_C, w_A_B)
    fbc = fbc2.reshape(N * NP)
    fab = fab2.reshape(N * NP)
    return _build_gather_combine()(ia, ib, ic, fbc, fab)


# speedup vs baseline: 4.2497x; 1.2077x over previous
import functools

import jax
import jax.numpy as jnp
from jax import lax
from jax.experimental import pallas as pl
from jax.experimental.pallas import tpu as pltpu
from jax.experimental.pallas import tpu_sc as plsc

N = 1000
NP = 1024
BQ = 16384
NC, NS, L = 2, 16, 16
NW = NC * NS
BPW = BQ // NW
CHUNK = 128
NCHUNK = BPW // CHUNK


GR = 200


def _fold_body(wcr_ref, wcc_ref, wbc_ref, wab_ref, fbc_ref, fab_ref):
    wbc = wbc_ref[...]
    m1 = jnp.max(wbc, axis=1, keepdims=True)
    lse_bc = m1 + jnp.log(jnp.sum(jnp.exp(wbc - m1), axis=1, keepdims=True))
    wab = wab_ref[...]
    m2 = jnp.max(wab, axis=1, keepdims=True)
    lse_ab = m2 + jnp.log(jnp.sum(jnp.exp(wab - m2), axis=1, keepdims=True))
    wcr = wcr_ref[...]
    mc = jnp.max(wcr)
    lse_c = mc + jnp.log(jnp.sum(jnp.exp(wcr - mc)))
    adj_bc = wbc + (wcc_ref[...] - lse_c - lse_bc)
    adj_ab = wab - lse_ab
    for k in range(GR):
        fbc_ref[pl.ds(k * NP, N)] = adj_bc[k, :]
        fab_ref[pl.ds(k * NP, N)] = adj_ab[k, :]


_fold_call = pl.pallas_call(
    _fold_body,
    grid=(N // GR,),
    in_specs=[
        pl.BlockSpec((1, N), lambda i: (0, 0)),
        pl.BlockSpec((GR, 1), lambda i: (i, 0)),
        pl.BlockSpec((GR, N), lambda i: (i, 0)),
        pl.BlockSpec((GR, N), lambda i: (i, 0)),
    ],
    out_specs=(
        pl.BlockSpec((GR * NP,), lambda i: (i,)),
        pl.BlockSpec((GR * NP,), lambda i: (i,)),
    ),
    out_shape=(
        jax.ShapeDtypeStruct((N * NP,), jnp.float32),
        jax.ShapeDtypeStruct((N * NP,), jnp.float32),
    ),
)


@functools.cache
def _build_gather_combine():
  mesh = plsc.VectorSubcoreMesh(core_axis_name="c", subcore_axis_name="s")

  @functools.partial(
      pl.kernel,
      out_type=jax.ShapeDtypeStruct((BQ,), jnp.float32),
      mesh=mesh,
      scratch_types=[
          pltpu.VMEM((BPW,), jnp.int32),
          pltpu.VMEM((BPW,), jnp.int32),
          pltpu.VMEM((BPW,), jnp.int32),
          pltpu.VMEM((NCHUNK, CHUNK), jnp.int32),
          pltpu.VMEM((NCHUNK, CHUNK), jnp.int32),
          pltpu.VMEM((BPW,), jnp.float32),
          pltpu.VMEM((BPW,), jnp.float32),
          pltpu.VMEM((BPW,), jnp.float32),
          pltpu.SemaphoreType.DMA,
      ],
  )
  def _gather_combine(ia_hbm, ib_hbm, ic_hbm, fbc_hbm, fab_hbm,
                      out_hbm, ia_v, ib_v, ic_v, xbc_v, xab_v,
                      gbc_v, gab_v, out_v, sem):
    wid = lax.axis_index("s") * NC + lax.axis_index("c")
    base = wid * BPW
    pltpu.sync_copy(ia_hbm.at[pl.ds(base, BPW)], ia_v)
    pltpu.sync_copy(ib_hbm.at[pl.ds(base, BPW)], ib_v)
    pltpu.sync_copy(ic_hbm.at[pl.ds(base, BPW)], ic_v)
    for j in range(BPW // L):
        r, o = j // (CHUNK // L), (j % (CHUNK // L)) * L
        sl = pl.ds(j * L, L)
        a16 = ia_v[sl]
        b16 = ib_v[sl]
        c16 = ic_v[sl]
        osl = pl.ds(o, L)
        xbc_v[r, osl] = c16 * NP + b16
        xab_v[r, osl] = b16 * NP + a16
    copies = []
    for r in range(NCHUNK):
        dsl = pl.ds(r * CHUNK, CHUNK)
        copies.append(pltpu.async_copy(
            fbc_hbm.at[xbc_v.at[r]], gbc_v.at[dsl], sem))
        copies.append(pltpu.async_copy(
            fab_hbm.at[xab_v.at[r]], gab_v.at[dsl], sem))
    for cp in copies:
        cp.wait()
    for j in range(BPW // L):
        sl = pl.ds(j * L, L)
        out_v[sl] = gbc_v[sl] + gab_v[sl]
    pltpu.sync_copy(out_v, out_hbm.at[pl.ds(base, BPW)])

  return _gather_combine


def kernel(inputs, w_C, w_B_C, w_A_B):
    idx = inputs.astype(jnp.int32)
    ia, ib, ic = idx[:, 0], idx[:, 1], idx[:, 2]
    fbc, fab = _fold_call(w_C.reshape(1, N), w_C.reshape(N, 1), w_B_C, w_A_B)
    return _build_gather_combine()(ia, ib, ic, fbc, fab)


# speedup vs baseline: 4.3735x; 1.0291x over previous
import functools

import jax
import jax.numpy as jnp
from jax import lax
from jax.experimental import pallas as pl
from jax.experimental.pallas import tpu as pltpu
from jax.experimental.pallas import tpu_sc as plsc

N = 1000
NP = 1024
BQ = 16384
NC, NS, L = 2, 16, 16
NW = NC * NS
BPW = BQ // NW
CHUNK = 128
NCHUNK = BPW // CHUNK


GR = 200


def _fold_body(wcr_ref, wcc_ref, wbc_ref, wab_ref, ia_ref, ib_ref, ic_ref,
               fbc_ref, fab_ref, xbc_ref, xab_ref):
    ib = ib_ref[...]
    xbc_ref[...] = ic_ref[...] * NP + ib
    xab_ref[...] = ib * NP + ia_ref[...]
    wbc = wbc_ref[...]
    m1 = jnp.max(wbc, axis=1, keepdims=True)
    lse_bc = m1 + jnp.log(jnp.sum(jnp.exp(wbc - m1), axis=1, keepdims=True))
    wab = wab_ref[...]
    m2 = jnp.max(wab, axis=1, keepdims=True)
    lse_ab = m2 + jnp.log(jnp.sum(jnp.exp(wab - m2), axis=1, keepdims=True))
    wcr = wcr_ref[...]
    mc = jnp.max(wcr)
    lse_c = mc + jnp.log(jnp.sum(jnp.exp(wcr - mc)))
    adj_bc = wbc + (wcc_ref[...] - lse_c - lse_bc)
    adj_ab = wab - lse_ab
    for k in range(GR):
        fbc_ref[pl.ds(k * NP, N)] = adj_bc[k, :]
        fab_ref[pl.ds(k * NP, N)] = adj_ab[k, :]


_fold_call = pl.pallas_call(
    _fold_body,
    grid=(N // GR,),
    in_specs=[
        pl.BlockSpec((1, N), lambda i: (0, 0)),
        pl.BlockSpec((GR, 1), lambda i: (i, 0)),
        pl.BlockSpec((GR, N), lambda i: (i, 0)),
        pl.BlockSpec((GR, N), lambda i: (i, 0)),
        pl.BlockSpec((BQ,), lambda i: (0,)),
        pl.BlockSpec((BQ,), lambda i: (0,)),
        pl.BlockSpec((BQ,), lambda i: (0,)),
    ],
    out_specs=(
        pl.BlockSpec((GR * NP,), lambda i: (i,)),
        pl.BlockSpec((GR * NP,), lambda i: (i,)),
        pl.BlockSpec((BQ,), lambda i: (0,)),
        pl.BlockSpec((BQ,), lambda i: (0,)),
    ),
    out_shape=(
        jax.ShapeDtypeStruct((N * NP,), jnp.float32),
        jax.ShapeDtypeStruct((N * NP,), jnp.float32),
        jax.ShapeDtypeStruct((BQ,), jnp.int32),
        jax.ShapeDtypeStruct((BQ,), jnp.int32),
    ),
)


@functools.cache
def _build_gather_combine():
  mesh = plsc.VectorSubcoreMesh(core_axis_name="c", subcore_axis_name="s")

  @functools.partial(
      pl.kernel,
      out_type=jax.ShapeDtypeStruct((BQ,), jnp.float32),
      mesh=mesh,
      scratch_types=[
          pltpu.VMEM((BPW,), jnp.int32),
          pltpu.VMEM((BPW,), jnp.int32),
          pltpu.VMEM((BPW,), jnp.float32),
          pltpu.VMEM((BPW,), jnp.float32),
          pltpu.VMEM((BPW,), jnp.float32),
          pltpu.SemaphoreType.DMA,
      ],
  )
  def _gather_combine(xbc_hbm, xab_hbm, fbc_hbm, fab_hbm,
                      out_hbm, xbc_v, xab_v, gbc_v, gab_v, out_v, sem):
    wid = lax.axis_index("s") * NC + lax.axis_index("c")
    base = wid * BPW
    c1 = pltpu.async_copy(xbc_hbm.at[pl.ds(base, BPW)], xbc_v, sem)
    c2 = pltpu.async_copy(xab_hbm.at[pl.ds(base, BPW)], xab_v, sem)
    c1.wait()
    c2.wait()
    copies = []
    for r in range(NCHUNK):
        dsl = pl.ds(r * CHUNK, CHUNK)
        copies.append(pltpu.async_copy(
            fbc_hbm.at[xbc_v.at[dsl]], gbc_v.at[dsl], sem))
        copies.append(pltpu.async_copy(
            fab_hbm.at[xab_v.at[dsl]], gab_v.at[dsl], sem))
    for cp in copies:
        cp.wait()
    for j in range(BPW // L):
        sl = pl.ds(j * L, L)
        out_v[sl] = gbc_v[sl] + gab_v[sl]
    pltpu.sync_copy(out_v, out_hbm.at[pl.ds(base, BPW)])

  return _gather_combine


def kernel(inputs, w_C, w_B_C, w_A_B):
    idx = inputs.astype(jnp.int32)
    ia, ib, ic = idx[:, 0], idx[:, 1], idx[:, 2]
    fbc, fab, xbc, xab = _fold_call(
        w_C.reshape(1, N), w_C.reshape(N, 1), w_B_C, w_A_B, ia, ib, ic)
    return _build_gather_combine()(xbc, xab, fbc, fab)


# speedup vs baseline: 4.3774x; 1.0009x over previous
import functools

import jax
import jax.numpy as jnp
from jax import lax
from jax.experimental import pallas as pl
from jax.experimental.pallas import tpu as pltpu
from jax.experimental.pallas import tpu_sc as plsc

N = 1000
NP = 1024
BQ = 16384
NC, NS, L = 2, 16, 16
NW = NC * NS
BPW = BQ // NW
CHUNK = 128
NCHUNK = BPW // CHUNK


GR = 200


def _fold_body(wcr_ref, wcc_ref, wbc_ref, wab_ref, ia_ref, ib_ref, ic_ref,
               fbc_ref, fab_ref, xbc_ref, xab_ref):
    ib = ib_ref[...]
    xbc_ref[...] = ic_ref[...] * NP + ib
    xab_ref[...] = ib * NP + ia_ref[...]
    wbc = wbc_ref[...]
    m1 = jnp.max(wbc, axis=1, keepdims=True)
    lse_bc = m1 + jnp.log(jnp.sum(jnp.exp(wbc - m1), axis=1, keepdims=True))
    wab = wab_ref[...]
    m2 = jnp.max(wab, axis=1, keepdims=True)
    lse_ab = m2 + jnp.log(jnp.sum(jnp.exp(wab - m2), axis=1, keepdims=True))
    wcr = wcr_ref[...]
    mc = jnp.max(wcr)
    lse_c = mc + jnp.log(jnp.sum(jnp.exp(wcr - mc)))
    adj_bc = wbc + (wcc_ref[...] - lse_c - lse_bc)
    adj_ab = wab - lse_ab
    for k in range(GR):
        fbc_ref[pl.ds(k * NP, N)] = adj_bc[k, :]
        fab_ref[pl.ds(k * NP, N)] = adj_ab[k, :]


_fold_call = pl.pallas_call(
    _fold_body,
    grid=(N // GR,),
    in_specs=[
        pl.BlockSpec((1, N), lambda i: (0, 0)),
        pl.BlockSpec((GR, 1), lambda i: (i, 0)),
        pl.BlockSpec((GR, N), lambda i: (i, 0)),
        pl.BlockSpec((GR, N), lambda i: (i, 0)),
        pl.BlockSpec((BQ,), lambda i: (0,)),
        pl.BlockSpec((BQ,), lambda i: (0,)),
        pl.BlockSpec((BQ,), lambda i: (0,)),
    ],
    out_specs=(
        pl.BlockSpec((GR * NP,), lambda i: (i,)),
        pl.BlockSpec((GR * NP,), lambda i: (i,)),
        pl.BlockSpec((BQ,), lambda i: (0,)),
        pl.BlockSpec((BQ,), lambda i: (0,)),
    ),
    out_shape=(
        jax.ShapeDtypeStruct((N * NP,), jnp.float32),
        jax.ShapeDtypeStruct((N * NP,), jnp.float32),
        jax.ShapeDtypeStruct((BQ,), jnp.int32),
        jax.ShapeDtypeStruct((BQ,), jnp.int32),
    ),
)


@functools.cache
def _build_gather_combine():
  mesh = plsc.VectorSubcoreMesh(core_axis_name="c", subcore_axis_name="s")

  @functools.partial(
      pl.kernel,
      out_type=jax.ShapeDtypeStruct((BQ,), jnp.float32),
      mesh=mesh,
      scratch_types=[
          pltpu.VMEM((BPW,), jnp.int32),
          pltpu.VMEM((BPW,), jnp.int32),
          pltpu.VMEM((BPW,), jnp.float32),
          pltpu.VMEM((BPW,), jnp.float32),
          pltpu.VMEM((BPW,), jnp.float32),
          pltpu.SemaphoreType.DMA,
      ],
  )
  def _gather_combine(xbc_hbm, xab_hbm, fbc_hbm, fab_hbm,
                      out_hbm, xbc_v, xab_v, gbc_v, gab_v, out_v, sem):
    wid = lax.axis_index("s") * NC + lax.axis_index("c")
    base = wid * BPW
    c1 = pltpu.async_copy(xbc_hbm.at[pl.ds(base, BPW)], xbc_v, sem)
    c2 = pltpu.async_copy(xab_hbm.at[pl.ds(base, BPW)], xab_v, sem)
    c1.wait()
    c2.wait()
    g1 = pltpu.async_copy(fbc_hbm.at[xbc_v], gbc_v, sem)
    g2 = pltpu.async_copy(fab_hbm.at[xab_v], gab_v, sem)
    g1.wait()
    g2.wait()

    @pl.loop(0, BPW // L)
    def _combine(j):
        sl = pl.ds(j * L, L)
        out_v[sl] = gbc_v[sl] + gab_v[sl]

    pltpu.sync_copy(out_v, out_hbm.at[pl.ds(base, BPW)])

  return _gather_combine


def kernel(inputs, w_C, w_B_C, w_A_B):
    idx = inputs.astype(jnp.int32)
    ia, ib, ic = idx[:, 0], idx[:, 1], idx[:, 2]
    fbc, fab, xbc, xab = _fold_call(
        w_C.reshape(1, N), w_C.reshape(N, 1), w_B_C, w_A_B, ia, ib, ic)
    return _build_gather_combine()(xbc, xab, fbc, fab)
